# Initial kernel scaffold; baseline (speedup 1.0000x reference)
import jax
import jax.numpy as jnp
from jax.experimental import pallas as pl


def kernel(inputs):
    raise NotImplementedError("write your pallas kernel here")


# speedup vs baseline: 1.4245x; 1.4245x over previous
import functools

import jax
import jax.numpy as jnp
from jax import lax
from jax.experimental import pallas as pl
from jax.experimental.pallas import tpu as pltpu
from jax.experimental.pallas import tpu_sc as plsc

DIM = 2048
UNIT_DIM = 64
N = 8192
BATCH = 16384
N_CHUNKS = DIM // UNIT_DIM

NUM_CORES = 2
NUM_SUBCORES = 16
NW = NUM_CORES * NUM_SUBCORES
B = BATCH * N_CHUNKS
B_PER_W = B // NW
K = 128
T = B_PER_W // K


def _gather_body(table_hbm, gidx_hbm, out_hbm, idx_v, rows_v, sem):
    wid = lax.axis_index("s") * NUM_CORES + lax.axis_index("c")
    base = wid * B_PER_W
    pltpu.sync_copy(gidx_hbm.at[wid], idx_v)

    def step(j, carry):
        pltpu.async_copy(table_hbm.at[idx_v.at[j]], rows_v, sem).wait()
        pltpu.sync_copy(rows_v, out_hbm.at[pl.ds(base + j * K, K)])
        return carry

    lax.fori_loop(0, T, step, 0)


@jax.jit
def _sc_gather(table, gidx):
    mesh = plsc.VectorSubcoreMesh(core_axis_name="c", subcore_axis_name="s")
    return pl.kernel(
        _gather_body,
        out_type=jax.ShapeDtypeStruct((B, UNIT_DIM), jnp.float32),
        mesh=mesh,
        scratch_types=[
            pltpu.VMEM((T, K), jnp.int32),
            pltpu.VMEM((K, UNIT_DIM), jnp.float32),
            pltpu.SemaphoreType.DMA,
        ],
        compiler_params=pltpu.CompilerParams(use_tc_tiling_on_sc=False),
    )(table, gidx)


def kernel(inputs):
    idx_key = jax.random.key(1)
    keys = jax.vmap(lambda i: jax.random.fold_in(idx_key, i))(jnp.arange(N_CHUNKS))
    idx = jax.vmap(lambda k: jax.random.randint(k, (BATCH,), 0, N))(keys)

    gidx = (idx.T * N_CHUNKS + jnp.arange(N_CHUNKS, dtype=jnp.int32)[None, :])
    gidx = gidx.reshape(NW, T, K)

    table = inputs.reshape(B, UNIT_DIM)
    out = _sc_gather(table, gidx)
    return out.reshape(BATCH, DIM)


# speedup vs baseline: 1.8257x; 1.2817x over previous
import jax
import jax.numpy as jnp
from jax import lax
from jax.experimental import pallas as pl
from jax.experimental.pallas import tpu as pltpu
from jax.experimental.pallas import tpu_sc as plsc

DIM = 2048
UNIT_DIM = 64
N = 8192
BATCH = 16384
N_CHUNKS = DIM // UNIT_DIM

NUM_CORES = 2
NUM_SUBCORES = 16
NW = NUM_CORES * NUM_SUBCORES
B = BATCH * N_CHUNKS
B_PER_W = B // NW
K = 128
T = B_PER_W // K
NBUF = 8
NROUNDS = T // NBUF


def _gather_body(table_hbm, gidx_hbm, out_hbm, idx_v, rows_v, *sems):
    gsem = sems[:NBUF]
    ssem = sems[NBUF:]
    wid = lax.axis_index("s") * NUM_CORES + lax.axis_index("c")
    base = wid * B_PER_W
    pltpu.sync_copy(gidx_hbm.at[wid], idx_v)

    for b in range(NBUF):
        pltpu.async_copy(table_hbm.at[idx_v.at[b]], rows_v.at[b], gsem[b])

    def do_slot(r, b, start_next):
        j = r * NBUF + b
        pltpu.make_async_copy(table_hbm.at[idx_v.at[b]], rows_v.at[b],
                              gsem[b]).wait()
        out_slice = out_hbm.at[pl.ds(base + j * K, K)]
        pltpu.async_copy(rows_v.at[b], out_slice, ssem[b])
        pltpu.make_async_copy(rows_v.at[b], out_slice, ssem[b]).wait()
        if start_next:
            pltpu.async_copy(table_hbm.at[idx_v.at[j + NBUF]], rows_v.at[b],
                             gsem[b])

    def round_body(r, carry):
        for b in range(NBUF):
            do_slot(r, b, True)
        return carry

    lax.fori_loop(0, NROUNDS - 1, round_body, 0)
    for b in range(NBUF):
        do_slot(NROUNDS - 1, b, False)


@jax.jit
def _sc_gather(table, gidx):
    mesh = plsc.VectorSubcoreMesh(core_axis_name="c", subcore_axis_name="s")
    return pl.kernel(
        _gather_body,
        out_type=jax.ShapeDtypeStruct((B, UNIT_DIM), jnp.float32),
        mesh=mesh,
        scratch_types=[
            pltpu.VMEM((T, K), jnp.int32),
            pltpu.VMEM((NBUF, K, UNIT_DIM), jnp.float32),
        ] + [pltpu.SemaphoreType.DMA] * (2 * NBUF),
        compiler_params=pltpu.CompilerParams(use_tc_tiling_on_sc=False),
    )(table, gidx)


def kernel(inputs):
    idx_key = jax.random.key(1)
    keys = jax.vmap(lambda i: jax.random.fold_in(idx_key, i))(jnp.arange(N_CHUNKS))
    idx = jax.vmap(lambda k: jax.random.randint(k, (BATCH,), 0, N))(keys)

    gidx = (idx.T * N_CHUNKS + jnp.arange(N_CHUNKS, dtype=jnp.int32)[None, :])
    gidx = gidx.reshape(NW, T, K)

    table = inputs[:N].reshape(N * N_CHUNKS, UNIT_DIM)
    out = _sc_gather(table, gidx)
    return out.reshape(BATCH, DIM)
